# initial kernel scaffold (unmeasured)
import jax
import jax.numpy as jnp
from jax import lax
from jax.experimental import pallas as pl
from jax.experimental.pallas import tpu as pltpu

N_DEV = 16
N_TOK = 512
D_IN = 256
D_OUT = 512
E_PER_DEV = 2
CHUNK = N_TOK // N_DEV


def kernel(x, router_W, route_idx, expert_W):
    def body(
        x_ref,
        rw_ref,
        idx_ref,
        w_ref,
        out_ref,
        partial_ref,
        ag_src_ref,
        rs_buf,
        ag_buf,
        rs_send,
        rs_recv,
        ag_send,
        ag_recv,
    ):
        me = lax.axis_index("i")

        barrier = pltpu.get_barrier_semaphore()
        for d in range(1, N_DEV):
            peer = lax.rem(me + d, N_DEV)
            pl.semaphore_signal(
                barrier,
                inc=1,
                device_id=(peer,),
                device_id_type=pl.DeviceIdType.MESH,
            )
        pl.semaphore_wait(barrier, N_DEV - 1)

        xv = x_ref[:, :].astype(jnp.bfloat16)
        acc = jnp.zeros((N_TOK, D_OUT), jnp.float32)
        for j in range(E_PER_DEV):
            eid = me * E_PER_DEV + j
            mask = idx_ref[:, :] == eid
            xm = jnp.where(mask, xv, jnp.zeros_like(xv))
            acc = acc + lax.dot_general(
                xm,
                w_ref[j].astype(jnp.bfloat16),
                (((1,), (0,)), ((), ())),
                preferred_element_type=jnp.float32,
            )
        partial_ref[:, :, :] = acc.astype(jnp.bfloat16).reshape(
            N_DEV, CHUNK, D_OUT
        )

        rs_rdmas = []
        for d in range(1, N_DEV):
            peer = lax.rem(me + d, N_DEV)
            rdma = pltpu.make_async_remote_copy(
                src_ref=partial_ref.at[peer],
                dst_ref=rs_buf.at[d - 1],
                send_sem=rs_send.at[d - 1],
                recv_sem=rs_recv.at[d - 1],
                device_id=(peer,),
                device_id_type=pl.DeviceIdType.MESH,
            )
            rdma.start()
            rs_rdmas.append(rdma)

        for r in rs_rdmas:
            r.wait_recv()
        chunk = lax.dynamic_slice_in_dim(acc, me * CHUNK, CHUNK, 0)
        for s in range(N_DEV - 1):
            chunk = chunk + rs_buf[s].astype(jnp.float32)
        out_ref[pl.ds(me * CHUNK, CHUNK), :] = chunk
        ag_src_ref[:, :] = chunk.astype(jnp.bfloat16)

        ag_rdmas = []
        for d in range(1, N_DEV):
            peer = lax.rem(me + d, N_DEV)
            rdma = pltpu.make_async_remote_copy(
                src_ref=ag_src_ref,
                dst_ref=ag_buf.at[d - 1],
                send_sem=ag_send.at[d - 1],
                recv_sem=ag_recv.at[d - 1],
                device_id=(peer,),
                device_id_type=pl.DeviceIdType.MESH,
            )
            rdma.start()
            ag_rdmas.append(rdma)

        for s, r in enumerate(ag_rdmas):
            r.wait_recv()
            src_dev = lax.rem(me + (N_DEV - 1 - s), N_DEV)
            out_ref[pl.ds(src_dev * CHUNK, CHUNK), :] = ag_buf[s].astype(
                jnp.float32
            )

        for r in rs_rdmas:
            r.wait_send()
        for r in ag_rdmas:
            r.wait_send()

    return pl.pallas_call(
        body,
        out_shape=jax.ShapeDtypeStruct((N_TOK, D_OUT), jnp.float32),
        in_specs=[pl.BlockSpec(memory_space=pltpu.VMEM)] * 4,
        out_specs=pl.BlockSpec(memory_space=pltpu.VMEM),
        scratch_shapes=[
            pltpu.VMEM((N_DEV, CHUNK, D_OUT), jnp.bfloat16),
            pltpu.VMEM((CHUNK, D_OUT), jnp.bfloat16),
            pltpu.VMEM((N_DEV - 1, CHUNK, D_OUT), jnp.bfloat16),
            pltpu.VMEM((N_DEV - 1, CHUNK, D_OUT), jnp.bfloat16),
            pltpu.SemaphoreType.DMA((N_DEV - 1,)),
            pltpu.SemaphoreType.DMA((N_DEV - 1,)),
            pltpu.SemaphoreType.DMA((N_DEV - 1,)),
            pltpu.SemaphoreType.DMA((N_DEV - 1,)),
        ],
        compiler_params=pltpu.CompilerParams(collective_id=0),
    )(x, router_W, route_idx, expert_W)


# baseline (device time: 24247 ns/iter reference)
import jax
import jax.numpy as jnp
from jax import lax
from jax.experimental import pallas as pl
from jax.experimental.pallas import tpu as pltpu

N_DEV = 16
N_TOK = 512
D_IN = 256
D_OUT = 512
E_PER_DEV = 2
CHUNK = N_TOK // N_DEV


def kernel(x, router_W, route_idx, expert_W):
    def body(
        x_ref,
        rw_ref,
        idx_ref,
        w_ref,
        out_ref,
        partial_ref,
        ag_src_ref,
        rs_buf,
        ag_buf,
        rs_send,
        rs_recv,
        ag_send,
        ag_recv,
    ):
        me = lax.axis_index("i")

        barrier = pltpu.get_barrier_semaphore()
        for d in range(1, N_DEV):
            peer = lax.rem(me + d, N_DEV)
            pl.semaphore_signal(
                barrier,
                inc=1,
                device_id=(peer,),
                device_id_type=pl.DeviceIdType.MESH,
            )
        pl.semaphore_wait(barrier, N_DEV - 1)

        xv = x_ref[:, :].astype(jnp.bfloat16)
        acc = jnp.zeros((N_TOK, D_OUT), jnp.float32)
        for j in range(E_PER_DEV):
            eid = me * E_PER_DEV + j
            mask = idx_ref[:, :] == eid
            xm = jnp.where(mask, xv, jnp.zeros_like(xv))
            acc = acc + lax.dot_general(
                xm,
                w_ref[j].astype(jnp.bfloat16),
                (((1,), (0,)), ((), ())),
                preferred_element_type=jnp.float32,
            )
        partial_ref[:, :, :] = acc.astype(jnp.bfloat16).reshape(
            N_DEV, CHUNK, D_OUT
        )

        rs_rdmas = []
        for d in range(1, N_DEV):
            peer = lax.rem(me + d, N_DEV)
            rdma = pltpu.make_async_remote_copy(
                src_ref=partial_ref.at[peer],
                dst_ref=rs_buf.at[d - 1],
                send_sem=rs_send.at[d - 1],
                recv_sem=rs_recv.at[d - 1],
                device_id=(peer,),
                device_id_type=pl.DeviceIdType.MESH,
            )
            rdma.start()
            rs_rdmas.append(rdma)

        for r in rs_rdmas:
            r.wait_recv()
        chunk = partial_ref[me].astype(jnp.float32)
        for s in range(N_DEV - 1):
            chunk = chunk + rs_buf[s].astype(jnp.float32)
        out_ref[pl.ds(me * CHUNK, CHUNK), :] = chunk
        ag_src_ref[:, :] = chunk.astype(jnp.bfloat16)

        ag_rdmas = []
        for d in range(1, N_DEV):
            peer = lax.rem(me + d, N_DEV)
            rdma = pltpu.make_async_remote_copy(
                src_ref=ag_src_ref,
                dst_ref=ag_buf.at[d - 1],
                send_sem=ag_send.at[d - 1],
                recv_sem=ag_recv.at[d - 1],
                device_id=(peer,),
                device_id_type=pl.DeviceIdType.MESH,
            )
            rdma.start()
            ag_rdmas.append(rdma)

        for s, r in enumerate(ag_rdmas):
            r.wait_recv()
            src_dev = lax.rem(me + (N_DEV - 1 - s), N_DEV)
            out_ref[pl.ds(src_dev * CHUNK, CHUNK), :] = ag_buf[s].astype(
                jnp.float32
            )

        for r in rs_rdmas:
            r.wait_send()
        for r in ag_rdmas:
            r.wait_send()

    return pl.pallas_call(
        body,
        out_shape=jax.ShapeDtypeStruct((N_TOK, D_OUT), jnp.float32),
        in_specs=[pl.BlockSpec(memory_space=pltpu.VMEM)] * 4,
        out_specs=pl.BlockSpec(memory_space=pltpu.VMEM),
        scratch_shapes=[
            pltpu.VMEM((N_DEV, CHUNK, D_OUT), jnp.bfloat16),
            pltpu.VMEM((CHUNK, D_OUT), jnp.bfloat16),
            pltpu.VMEM((N_DEV - 1, CHUNK, D_OUT), jnp.bfloat16),
            pltpu.VMEM((N_DEV - 1, CHUNK, D_OUT), jnp.bfloat16),
            pltpu.SemaphoreType.DMA((N_DEV - 1,)),
            pltpu.SemaphoreType.DMA((N_DEV - 1,)),
            pltpu.SemaphoreType.DMA((N_DEV - 1,)),
            pltpu.SemaphoreType.DMA((N_DEV - 1,)),
        ],
        compiler_params=pltpu.CompilerParams(collective_id=0),
    )(x, router_W, route_idx, expert_W)


# device time: 23696 ns/iter; 1.0233x vs baseline; 1.0233x over previous
import jax
import jax.numpy as jnp
from jax import lax
from jax.experimental import pallas as pl
from jax.experimental.pallas import tpu as pltpu

N_DEV = 16
N_TOK = 512
D_IN = 256
D_OUT = 512
E_PER_DEV = 2
CHUNK = N_TOK // N_DEV


def kernel(x, router_W, route_idx, expert_W):
    def body(
        x_ref,
        rw_ref,
        idx_ref,
        w_ref,
        out_ref,
        partial_ref,
        ag_src_ref,
        rs_buf,
        ag_buf,
        rs_send,
        rs_recv,
        ag_send,
        ag_recv,
    ):
        me = lax.axis_index("i")

        barrier = pltpu.get_barrier_semaphore()
        for d in range(1, N_DEV):
            peer = lax.rem(me + d, N_DEV)
            pl.semaphore_signal(
                barrier,
                inc=1,
                device_id=(peer,),
                device_id_type=pl.DeviceIdType.MESH,
            )

        xv = x_ref[:, :].astype(jnp.bfloat16)
        acc = jnp.zeros((N_TOK, D_OUT), jnp.float32)
        for j in range(E_PER_DEV):
            eid = me * E_PER_DEV + j
            mask = idx_ref[:, :] == eid
            xm = jnp.where(mask, xv, jnp.zeros_like(xv))
            acc = acc + lax.dot_general(
                xm,
                w_ref[j].astype(jnp.bfloat16),
                (((1,), (0,)), ((), ())),
                preferred_element_type=jnp.float32,
            )
        partial_ref[:, :, :] = acc.astype(jnp.bfloat16).reshape(
            N_DEV, CHUNK, D_OUT
        )
        pl.semaphore_wait(barrier, N_DEV - 1)

        rs_rdmas = []
        for d in range(1, N_DEV):
            peer = lax.rem(me + d, N_DEV)
            rdma = pltpu.make_async_remote_copy(
                src_ref=partial_ref.at[peer],
                dst_ref=rs_buf.at[d - 1],
                send_sem=rs_send.at[d - 1],
                recv_sem=rs_recv.at[d - 1],
                device_id=(peer,),
                device_id_type=pl.DeviceIdType.MESH,
            )
            rdma.start()
            rs_rdmas.append(rdma)

        chunk = partial_ref[me].astype(jnp.float32)
        for s, r in enumerate(rs_rdmas):
            r.wait_recv()
            chunk = chunk + rs_buf[s].astype(jnp.float32)
        ag_src_ref[:, :] = chunk.astype(jnp.bfloat16)

        ag_rdmas = []
        for d in range(1, N_DEV):
            peer = lax.rem(me + d, N_DEV)
            rdma = pltpu.make_async_remote_copy(
                src_ref=ag_src_ref,
                dst_ref=ag_buf.at[d - 1],
                send_sem=ag_send.at[d - 1],
                recv_sem=ag_recv.at[d - 1],
                device_id=(peer,),
                device_id_type=pl.DeviceIdType.MESH,
            )
            rdma.start()
            ag_rdmas.append(rdma)

        out_ref[pl.ds(me * CHUNK, CHUNK), :] = chunk

        for s, r in enumerate(ag_rdmas):
            r.wait_recv()
            src_dev = lax.rem(me + (N_DEV - 1 - s), N_DEV)
            out_ref[pl.ds(src_dev * CHUNK, CHUNK), :] = ag_buf[s].astype(
                jnp.float32
            )

        for r in rs_rdmas:
            r.wait_send()
        for r in ag_rdmas:
            r.wait_send()

    return pl.pallas_call(
        body,
        out_shape=jax.ShapeDtypeStruct((N_TOK, D_OUT), jnp.float32),
        in_specs=[pl.BlockSpec(memory_space=pltpu.VMEM)] * 4,
        out_specs=pl.BlockSpec(memory_space=pltpu.VMEM),
        scratch_shapes=[
            pltpu.VMEM((N_DEV, CHUNK, D_OUT), jnp.bfloat16),
            pltpu.VMEM((CHUNK, D_OUT), jnp.bfloat16),
            pltpu.VMEM((N_DEV - 1, CHUNK, D_OUT), jnp.bfloat16),
            pltpu.VMEM((N_DEV - 1, CHUNK, D_OUT), jnp.bfloat16),
            pltpu.SemaphoreType.DMA((N_DEV - 1,)),
            pltpu.SemaphoreType.DMA((N_DEV - 1,)),
            pltpu.SemaphoreType.DMA((N_DEV - 1,)),
            pltpu.SemaphoreType.DMA((N_DEV - 1,)),
        ],
        compiler_params=pltpu.CompilerParams(collective_id=0),
    )(x, router_W, route_idx, expert_W)
